# Initial kernel scaffold; baseline (speedup 1.0000x reference)
import jax
import jax.numpy as jnp
from jax.experimental import pallas as pl


def kernel(node_feat, src, dst, W0, b0, g0, be0, W1, b1, g1, be1, W2, b2):
    raise NotImplementedError("write your pallas kernel here")


# speedup vs baseline: 4.2991x; 4.2991x over previous
import functools

import jax
import jax.numpy as jnp
from jax import lax
from jax.experimental import pallas as pl
from jax.experimental.pallas import tpu as pltpu
from jax.experimental.pallas import tpu_sc as plsc

N = 10000
E = 160000
NPAD = 10240
EPAD = 163840
DUMMY = 10000
B = 128
NB = 80
TROWS = 640
MT = 1000
GRID = N // MT

_mesh = plsc.VectorSubcoreMesh(core_axis_name="c", subcore_axis_name="s")


def _make_scatter(n_tables):
    n_pass = n_tables // 2

    @functools.partial(
        pl.kernel, mesh=_mesh,
        out_type=[jax.ShapeDtypeStruct((NPAD, 128), jnp.float32)] * n_tables,
        scratch_types=[
            pltpu.VMEM((B,), jnp.int32),
            pltpu.VMEM((B,), jnp.int32),
            pltpu.VMEM((B, 128), jnp.float32),
            pltpu.VMEM_SHARED((NPAD, 128), jnp.float32),
            pltpu.SemaphoreType.DMA,
        ],
    )
    def k(*refs):
        tables = refs[:n_tables]
        src_r = refs[n_tables]
        dst_r = refs[n_tables + 1]
        zr = refs[n_tables + 2]
        outs = refs[n_tables + 3: 2 * n_tables + 3]
        sidx, didx, rows, acc, sem = refs[2 * n_tables + 3:]
        cid = lax.axis_index("c")
        sid = lax.axis_index("s")

        def one_pass(table, out):
            for j in range(TROWS // B):
                pltpu.sync_copy(zr, acc.at[pl.ds(sid * TROWS + j * B, B)])
            plsc.subcore_barrier()

            def body(i, carry):
                base = sid * (NB * B) + i * B
                pltpu.sync_copy(src_r.at[pl.ds(base, B)], sidx)
                pltpu.sync_copy(dst_r.at[pl.ds(base, B)], didx)
                pltpu.async_copy(table.at[sidx], rows, sem).wait()
                pltpu.sync_copy(rows, acc.at[didx], add=True)
                return carry

            lax.fori_loop(0, NB, body, jnp.int32(0))
            plsc.subcore_barrier()
            for j in range(TROWS // B):
                r = pl.ds(sid * TROWS + j * B, B)
                pltpu.sync_copy(acc.at[r], out.at[r])

        @pl.when(cid == 0)
        def _():
            for p in range(n_pass):
                one_pass(tables[p], outs[p])

        @pl.when(cid == 1)
        def _():
            for p in range(n_pass):
                one_pass(tables[n_pass + p], outs[n_pass + p])

    return k


_scatter4 = _make_scatter(4)
_scatter2 = _make_scatter(2)


@functools.partial(
    pl.kernel, mesh=_mesh,
    out_type=[jax.ShapeDtypeStruct((NPAD, 128), jnp.float32)] * 2,
    scratch_types=[
        pltpu.VMEM((B,), jnp.int32),
        pltpu.VMEM((B, 128), jnp.float32),
        pltpu.VMEM_SHARED((NPAD, 128), jnp.float32),
    ],
)
def _deg_k(dst_r, ones_r, zrow_r, out_a, out_b, didx, ones_v, acc):
    cid = lax.axis_index("c")
    sid = lax.axis_index("s")
    pltpu.sync_copy(ones_r, ones_v)
    for j in range(TROWS // B):
        pltpu.sync_copy(zrow_r, acc.at[pl.ds(sid * TROWS + j * B, B)])
    plsc.subcore_barrier()

    def body(i, carry):
        base = cid * (EPAD // 2) + sid * (EPAD // 32) + i * B
        pltpu.sync_copy(dst_r.at[pl.ds(base, B)], didx)
        pltpu.sync_copy(ones_v, acc.at[didx], add=True)
        return carry

    lax.fori_loop(0, EPAD // 32 // B, body, jnp.int32(0))
    plsc.subcore_barrier()

    @pl.when(cid == 0)
    def _():
        for j in range(TROWS // B):
            r = pl.ds(sid * TROWS + j * B, B)
            pltpu.sync_copy(acc.at[r], out_a.at[r])

    @pl.when(cid == 1)
    def _():
        for j in range(TROWS // B):
            r = pl.ds(sid * TROWS + j * B, B)
            pltpu.sync_copy(acc.at[r], out_b.at[r])


def _mm_first(x, W, dinv):
    kdim = x.shape[1]

    def body(x_ref, w_ref, dv_ref, o0, o1, o2, o3):
        h = jnp.dot(x_ref[...], w_ref[...], preferred_element_type=jnp.float32)
        y = h * dv_ref[...]
        o0[...] = y[:, 0:128]
        o1[...] = y[:, 128:256]
        o2[...] = y[:, 256:384]
        o3[...] = y[:, 384:512]

    return pl.pallas_call(
        body, grid=(GRID,),
        in_specs=[
            pl.BlockSpec((MT, kdim), lambda i: (i, 0)),
            pl.BlockSpec((kdim, 512), lambda i: (0, 0)),
            pl.BlockSpec((MT, 1), lambda i: (i, 0)),
        ],
        out_specs=[pl.BlockSpec((MT, 128), lambda i: (i, 0))] * 4,
        out_shape=[jax.ShapeDtypeStruct((N, 128), jnp.float32)] * 4,
    )(x, W, dinv)


def _affine_stats(ss, ts, dinv, b):
    n_sl = len(ss)
    C = 128 * n_sl

    def body(*refs):
        s_refs = refs[:n_sl]
        t_refs = refs[n_sl:2 * n_sl]
        dv_ref = refs[2 * n_sl]
        b_ref = refs[2 * n_sl + 1]
        z_ref = refs[2 * n_sl + 2]
        st_ref = refs[2 * n_sl + 3]
        i = pl.program_id(0)
        s = jnp.concatenate([r[...] for r in s_refs], axis=1)
        t = jnp.concatenate([r[...] for r in t_refs], axis=1)
        z = dv_ref[...] * (s + t) + b_ref[...][None, :]
        z_ref[...] = z
        ps = jnp.stack([jnp.sum(z, axis=0), jnp.sum(z * z, axis=0)])

        @pl.when(i == 0)
        def _():
            st_ref[...] = ps

        @pl.when(i > 0)
        def _():
            st_ref[...] = st_ref[...] + ps

    return pl.pallas_call(
        body, grid=(GRID,),
        in_specs=(
            [pl.BlockSpec((MT, 128), lambda i: (i, 0))] * n_sl
            + [pl.BlockSpec((MT, 128), lambda i: (i, 0))] * n_sl
            + [pl.BlockSpec((MT, 1), lambda i: (i, 0)),
               pl.BlockSpec((C,), lambda i: (0,))]
        ),
        out_specs=[pl.BlockSpec((MT, C), lambda i: (i, 0)),
                   pl.BlockSpec((2, C), lambda i: (0, 0))],
        out_shape=[jax.ShapeDtypeStruct((N, C), jnp.float32),
                   jax.ShapeDtypeStruct((2, C), jnp.float32)],
    )(*ss, *ts, dinv, b)


def _bn_mm(z, st, g, be, W, dinv, n_out):
    C = z.shape[1]

    def body(z_ref, st_ref, g_ref, be_ref, w_ref, dv_ref, *outs):
        stv = st_ref[...]
        mean = stv[0] * (1.0 / N)
        var = stv[1] * (1.0 / N) - mean * mean
        scale = jax.lax.rsqrt(var + 1e-5) * g_ref[...]
        xn = (z_ref[...] - mean[None, :]) * scale[None, :] + be_ref[...][None, :]
        xn = jnp.maximum(xn, 0.0)
        h = jnp.dot(xn, w_ref[...], preferred_element_type=jnp.float32)
        h = h * dv_ref[...]
        for j, o in enumerate(outs):
            o[...] = h[:, j * 128:(j + 1) * 128]

    return pl.pallas_call(
        body, grid=(GRID,),
        in_specs=[
            pl.BlockSpec((MT, C), lambda i: (i, 0)),
            pl.BlockSpec((2, C), lambda i: (0, 0)),
            pl.BlockSpec((C,), lambda i: (0,)),
            pl.BlockSpec((C,), lambda i: (0,)),
            pl.BlockSpec((C, 128 * n_out), lambda i: (0, 0)),
            pl.BlockSpec((MT, 1), lambda i: (i, 0)),
        ],
        out_specs=[pl.BlockSpec((MT, 128), lambda i: (i, 0))] * n_out,
        out_shape=[jax.ShapeDtypeStruct((N, 128), jnp.float32)] * n_out,
    )(z, st, g, be, W, dinv)


def _final(ss, ts, dinv, b):
    n_sl = len(ss)
    C = 128 * n_sl

    def body(*refs):
        s_refs = refs[:n_sl]
        t_refs = refs[n_sl:2 * n_sl]
        dv_ref = refs[2 * n_sl]
        b_ref = refs[2 * n_sl + 1]
        o_ref = refs[2 * n_sl + 2]
        s = jnp.concatenate([r[...] for r in s_refs], axis=1)
        t = jnp.concatenate([r[...] for r in t_refs], axis=1)
        o_ref[...] = dv_ref[...] * (s + t) + b_ref[...][None, :]

    return pl.pallas_call(
        body, grid=(GRID,),
        in_specs=(
            [pl.BlockSpec((MT, 128), lambda i: (i, 0))] * (2 * n_sl)
            + [pl.BlockSpec((MT, 1), lambda i: (i, 0)),
               pl.BlockSpec((C,), lambda i: (0,))]
        ),
        out_specs=pl.BlockSpec((MT, C), lambda i: (i, 0)),
        out_shape=jax.ShapeDtypeStruct((N, C), jnp.float32),
    )(*ss, *ts, dinv, b)


def kernel(node_feat, src, dst, W0, b0, g0, be0, W1, b1, g1, be1, W2, b2):
    src32 = src.astype(jnp.int32)
    dst32 = dst.astype(jnp.int32)
    src_p = jnp.concatenate([src32, jnp.zeros((EPAD - E,), jnp.int32)])
    dst_p = jnp.concatenate([dst32, jnp.full((EPAD - E,), DUMMY, jnp.int32)])
    ones128 = jnp.ones((B, 128), jnp.float32)
    zrow = jnp.zeros((B, 128), jnp.float32)

    deg_a, deg_b = _deg_k(dst_p, ones128, zrow)
    dinv = jax.lax.rsqrt(deg_a[:N, :1] + deg_b[:N, :1] + 1.0)

    t = _mm_first(node_feat, W0, dinv)
    s = _scatter4(*t, src_p, dst_p, zrow)
    z, st = _affine_stats(s, t, dinv, b0)

    t1 = _bn_mm(z, st, g0, be0, W1, dinv, 4)
    s1 = _scatter4(*t1, src_p, dst_p, zrow)
    z1, st1 = _affine_stats(s1, t1, dinv, b1)

    t2 = _bn_mm(z1, st1, g1, be1, W2, dinv, 2)
    s2 = _scatter2(*t2, src_p, dst_p, zrow)
    return _final(s2, t2, dinv, b2)


# speedup vs baseline: 6.6768x; 1.5531x over previous
import functools

import jax
import jax.numpy as jnp
from jax import lax
from jax.experimental import pallas as pl
from jax.experimental.pallas import tpu as pltpu
from jax.experimental.pallas import tpu_sc as plsc

N = 10000
E = 160000
NPAD = 10240
EPAD = 163840
DUMMY = 10000
B = 128
NB = 80
TROWS = 640
MT = 1000
GRID = N // MT

_mesh = plsc.VectorSubcoreMesh(core_axis_name="c", subcore_axis_name="s")


def _make_scatter(n_tables):
    n_pass = n_tables // 2

    @functools.partial(
        pl.kernel, mesh=_mesh,
        out_type=[jax.ShapeDtypeStruct((NPAD, 128), jnp.float32)] * n_tables,
        scratch_types=[
            pltpu.VMEM((NB, B), jnp.int32),
            pltpu.VMEM((2, B), jnp.int32),
            pltpu.VMEM((2, B, 128), jnp.float32),
            pltpu.VMEM_SHARED((NPAD, 128), jnp.float32),
            pltpu.SemaphoreType.DMA,
            pltpu.SemaphoreType.DMA,
            pltpu.SemaphoreType.DMA,
            pltpu.SemaphoreType.DMA,
        ],
    )
    def k(*refs):
        tables = refs[:n_tables]
        src_r = refs[n_tables]
        dst_r = refs[n_tables + 1]
        zr = refs[n_tables + 2]
        outs = refs[n_tables + 3: 2 * n_tables + 3]
        sidx, didx, rows, acc, sem0, sem1, sem2, sem3 = refs[2 * n_tables + 3:]
        sems = (sem0, sem1)
        dsems = (sem2, sem3)
        cid = lax.axis_index("c")
        sid = lax.axis_index("s")

        pltpu.sync_copy(src_r.at[pl.ds(sid * NB, NB)], sidx)

        def one_pass(table, out):
            def issue(i, p):
                pltpu.async_copy(table.at[sidx.at[i]], rows.at[p], sems[p])
                pltpu.async_copy(dst_r.at[sid * NB + i], didx.at[p], dsems[p])

            for j in range(TROWS // B):
                pltpu.sync_copy(zr, acc.at[pl.ds(sid * TROWS + j * B, B)])
            plsc.subcore_barrier()

            issue(0, 0)

            def body(it, carry):
                for b in range(2):
                    i = 2 * it + b
                    issue(lax.rem(i + 1, NB), 1 - b)
                    pltpu.make_async_copy(
                        table.at[sidx.at[i]], rows.at[b], sems[b]).wait()
                    pltpu.make_async_copy(
                        dst_r.at[sid * NB + i], didx.at[b], dsems[b]).wait()
                    pltpu.sync_copy(rows.at[b], acc.at[didx.at[b]], add=True)
                return carry

            lax.fori_loop(0, NB // 2, body, jnp.int32(0))
            pltpu.make_async_copy(
                table.at[sidx.at[0]], rows.at[0], sem0).wait()
            pltpu.make_async_copy(
                dst_r.at[sid * NB], didx.at[0], sem2).wait()
            plsc.subcore_barrier()
            for j in range(TROWS // B):
                r = pl.ds(sid * TROWS + j * B, B)
                pltpu.sync_copy(acc.at[r], out.at[r])

        @pl.when(cid == 0)
        def _():
            for p in range(n_pass):
                one_pass(tables[p], outs[p])

        @pl.when(cid == 1)
        def _():
            for p in range(n_pass):
                one_pass(tables[n_pass + p], outs[n_pass + p])

    return k


_scatter4 = _make_scatter(4)
_scatter2 = _make_scatter(2)


@functools.partial(
    pl.kernel, mesh=_mesh,
    out_type=[jax.ShapeDtypeStruct((NPAD, 128), jnp.float32)] * 2,
    scratch_types=[
        pltpu.VMEM((EPAD // 32 // B, B), jnp.int32),
        pltpu.VMEM((B, 128), jnp.float32),
        pltpu.VMEM_SHARED((NPAD, 128), jnp.float32),
    ],
)
def _deg_k(dst_r, ones_r, zrow_r, out_a, out_b, didx, ones_v, acc):
    nb = EPAD // 32 // B
    cid = lax.axis_index("c")
    sid = lax.axis_index("s")
    pltpu.sync_copy(ones_r, ones_v)
    pltpu.sync_copy(dst_r.at[pl.ds(cid * (EPAD // B // 2) + sid * nb, nb)], didx)
    for j in range(TROWS // B):
        pltpu.sync_copy(zrow_r, acc.at[pl.ds(sid * TROWS + j * B, B)])
    plsc.subcore_barrier()

    def body(i, carry):
        pltpu.sync_copy(ones_v, acc.at[didx.at[i]], add=True)
        return carry

    lax.fori_loop(0, nb, body, jnp.int32(0))
    plsc.subcore_barrier()

    @pl.when(cid == 0)
    def _():
        for j in range(TROWS // B):
            r = pl.ds(sid * TROWS + j * B, B)
            pltpu.sync_copy(acc.at[r], out_a.at[r])

    @pl.when(cid == 1)
    def _():
        for j in range(TROWS // B):
            r = pl.ds(sid * TROWS + j * B, B)
            pltpu.sync_copy(acc.at[r], out_b.at[r])


def _mm_first(x, W, dinv):
    kdim = x.shape[1]

    def body(x_ref, w_ref, dv_ref, o0, o1, o2, o3):
        h = jnp.dot(x_ref[...], w_ref[...], preferred_element_type=jnp.float32)
        y = h * dv_ref[...]
        o0[...] = y[:, 0:128]
        o1[...] = y[:, 128:256]
        o2[...] = y[:, 256:384]
        o3[...] = y[:, 384:512]

    return pl.pallas_call(
        body, grid=(GRID,),
        in_specs=[
            pl.BlockSpec((MT, kdim), lambda i: (i, 0)),
            pl.BlockSpec((kdim, 512), lambda i: (0, 0)),
            pl.BlockSpec((MT, 1), lambda i: (i, 0)),
        ],
        out_specs=[pl.BlockSpec((MT, 128), lambda i: (i, 0))] * 4,
        out_shape=[jax.ShapeDtypeStruct((N, 128), jnp.float32)] * 4,
    )(x, W, dinv)


def _affine_stats(ss, ts, dinv, b):
    n_sl = len(ss)
    C = 128 * n_sl

    def body(*refs):
        s_refs = refs[:n_sl]
        t_refs = refs[n_sl:2 * n_sl]
        dv_ref = refs[2 * n_sl]
        b_ref = refs[2 * n_sl + 1]
        z_ref = refs[2 * n_sl + 2]
        st_ref = refs[2 * n_sl + 3]
        i = pl.program_id(0)
        s = jnp.concatenate([r[...] for r in s_refs], axis=1)
        t = jnp.concatenate([r[...] for r in t_refs], axis=1)
        z = dv_ref[...] * (s + t) + b_ref[...][None, :]
        z_ref[...] = z
        ps = jnp.stack([jnp.sum(z, axis=0), jnp.sum(z * z, axis=0)])

        @pl.when(i == 0)
        def _():
            st_ref[...] = ps

        @pl.when(i > 0)
        def _():
            st_ref[...] = st_ref[...] + ps

    return pl.pallas_call(
        body, grid=(GRID,),
        in_specs=(
            [pl.BlockSpec((MT, 128), lambda i: (i, 0))] * n_sl
            + [pl.BlockSpec((MT, 128), lambda i: (i, 0))] * n_sl
            + [pl.BlockSpec((MT, 1), lambda i: (i, 0)),
               pl.BlockSpec((C,), lambda i: (0,))]
        ),
        out_specs=[pl.BlockSpec((MT, C), lambda i: (i, 0)),
                   pl.BlockSpec((2, C), lambda i: (0, 0))],
        out_shape=[jax.ShapeDtypeStruct((N, C), jnp.float32),
                   jax.ShapeDtypeStruct((2, C), jnp.float32)],
    )(*ss, *ts, dinv, b)


def _bn_mm(z, st, g, be, W, dinv, n_out):
    C = z.shape[1]

    def body(z_ref, st_ref, g_ref, be_ref, w_ref, dv_ref, *outs):
        stv = st_ref[...]
        mean = stv[0] * (1.0 / N)
        var = stv[1] * (1.0 / N) - mean * mean
        scale = jax.lax.rsqrt(var + 1e-5) * g_ref[...]
        xn = (z_ref[...] - mean[None, :]) * scale[None, :] + be_ref[...][None, :]
        xn = jnp.maximum(xn, 0.0)
        h = jnp.dot(xn, w_ref[...], preferred_element_type=jnp.float32)
        h = h * dv_ref[...]
        for j, o in enumerate(outs):
            o[...] = h[:, j * 128:(j + 1) * 128]

    return pl.pallas_call(
        body, grid=(GRID,),
        in_specs=[
            pl.BlockSpec((MT, C), lambda i: (i, 0)),
            pl.BlockSpec((2, C), lambda i: (0, 0)),
            pl.BlockSpec((C,), lambda i: (0,)),
            pl.BlockSpec((C,), lambda i: (0,)),
            pl.BlockSpec((C, 128 * n_out), lambda i: (0, 0)),
            pl.BlockSpec((MT, 1), lambda i: (i, 0)),
        ],
        out_specs=[pl.BlockSpec((MT, 128), lambda i: (i, 0))] * n_out,
        out_shape=[jax.ShapeDtypeStruct((N, 128), jnp.float32)] * n_out,
    )(z, st, g, be, W, dinv)


def _final(ss, ts, dinv, b):
    n_sl = len(ss)
    C = 128 * n_sl

    def body(*refs):
        s_refs = refs[:n_sl]
        t_refs = refs[n_sl:2 * n_sl]
        dv_ref = refs[2 * n_sl]
        b_ref = refs[2 * n_sl + 1]
        o_ref = refs[2 * n_sl + 2]
        s = jnp.concatenate([r[...] for r in s_refs], axis=1)
        t = jnp.concatenate([r[...] for r in t_refs], axis=1)
        o_ref[...] = dv_ref[...] * (s + t) + b_ref[...][None, :]

    return pl.pallas_call(
        body, grid=(GRID,),
        in_specs=(
            [pl.BlockSpec((MT, 128), lambda i: (i, 0))] * (2 * n_sl)
            + [pl.BlockSpec((MT, 1), lambda i: (i, 0)),
               pl.BlockSpec((C,), lambda i: (0,))]
        ),
        out_specs=pl.BlockSpec((MT, C), lambda i: (i, 0)),
        out_shape=jax.ShapeDtypeStruct((N, C), jnp.float32),
    )(*ss, *ts, dinv, b)


def kernel(node_feat, src, dst, W0, b0, g0, be0, W1, b1, g1, be1, W2, b2):
    src32 = src.astype(jnp.int32)
    dst32 = dst.astype(jnp.int32)
    src_p = jnp.concatenate(
        [src32, jnp.zeros((EPAD - E,), jnp.int32)]).reshape(EPAD // B, B)
    dst_p = jnp.concatenate(
        [dst32, jnp.full((EPAD - E,), DUMMY, jnp.int32)]).reshape(EPAD // B, B)
    ones128 = jnp.ones((B, 128), jnp.float32)
    zrow = jnp.zeros((B, 128), jnp.float32)

    deg_a, deg_b = _deg_k(dst_p, ones128, zrow)
    dinv = jax.lax.rsqrt(deg_a[:N, :1] + deg_b[:N, :1] + 1.0)

    t = _mm_first(node_feat, W0, dinv)
    s = _scatter4(*t, src_p, dst_p, zrow)
    z, st = _affine_stats(s, t, dinv, b0)

    t1 = _bn_mm(z, st, g0, be0, W1, dinv, 4)
    s1 = _scatter4(*t1, src_p, dst_p, zrow)
    z1, st1 = _affine_stats(s1, t1, dinv, b1)

    t2 = _bn_mm(z1, st1, g1, be1, W2, dinv, 2)
    s2 = _scatter2(*t2, src_p, dst_p, zrow)
    return _final(s2, t2, dinv, b2)
